# baseline (speedup 1.0000x reference)
import functools

import numpy as np
import jax
import jax.numpy as jnp
from jax import lax
from jax.experimental import pallas as pl
from jax.experimental.pallas import tpu as pltpu
from jax.experimental.pallas import tpu_sc as plsc

B, N_NUM, N_CAT, D = 16384, 13, 26, 32
NF = N_NUM + N_CAT
WN, WC, WT = N_NUM * D, N_CAT * D, NF * D

NC, NS = 2, 16
NW = NC * NS
ROWS = B * N_CAT
CHUNK = 128
CPW = ROWS // (NW * CHUNK)
ROWS_PW = ROWS // NW


def _sc_gather(table, idx_flat):
    idx2d = idx_flat.reshape(ROWS // CHUNK, CHUNK)
    mesh = plsc.VectorSubcoreMesh(
        core_axis_name="c", subcore_axis_name="s", num_cores=NC, num_subcores=NS
    )

    @functools.partial(
        pl.kernel,
        out_type=jax.ShapeDtypeStruct((ROWS, D), jnp.float32),
        mesh=mesh,
        scratch_types=[
            pltpu.VMEM((CPW, CHUNK), jnp.int32),
            pltpu.VMEM((CHUNK, D), jnp.float32),
            pltpu.SemaphoreType.DMA,
        ],
        compiler_params=pltpu.CompilerParams(use_tc_tiling_on_sc=False),
    )
    def gather_k(idx_hbm, table_hbm, out_hbm, idx_v, rows_v, sem):
        wid = lax.axis_index("s") * NC + lax.axis_index("c")
        pltpu.sync_copy(idx_hbm.at[pl.ds(wid * CPW, CPW)], idx_v)
        out_rows = out_hbm
        base = wid * ROWS_PW

        def step(j, carry):
            pltpu.async_copy(table_hbm.at[idx_v.at[j]], rows_v, sem).wait()
            pltpu.sync_copy(rows_v, out_rows.at[pl.ds(base + j * CHUNK, CHUNK)])
            return carry

        lax.fori_loop(0, CPW, step, 0)

    return gather_k(idx2d, table)


def _np_f32(a):
    return np.ascontiguousarray(np.asarray(a, dtype=np.float32))


def _build_mats():
    l_t = np.arange(WT)
    g_t = np.arange(NF)
    Gs = (l_t[:, None] // D == g_t[None, :]).astype(np.float32) / D
    Gb = (g_t[:, None] == l_t[None, :] // D).astype(np.float32)
    Pxn = np.zeros((NF, NF), np.float32)
    for f in range(N_NUM):
        Pxn[3 * f, f] = 1.0
    Pidx = np.zeros((N_CAT, NF), np.float32)
    for f in range(N_CAT):
        Pidx[f, N_NUM + f] = 1.0
    Pmdn = np.zeros((NF, 2 * NF), np.float32)
    for f in range(N_NUM):
        Pmdn[3 * f + 1, 2 * f] = 1.0
        Pmdn[3 * f + 2, 2 * f + 1] = 1.0
    Smdc = np.zeros((2 * N_CAT, 2 * NF), np.float32)
    for j in range(2 * N_CAT):
        Smdc[j, 2 * N_NUM + j] = 1.0
    Esgn = np.zeros((N_CAT, 2 * NF), np.float32)
    for f in range(N_CAT):
        Esgn[f, 2 * N_NUM + 2 * f] = 1.0
    return tuple(_np_f32(m) for m in (Gs, Gb, Pxn, Pidx, Pmdn, Smdc, Esgn))


_MATS = _build_mats()

TB = 1024
_MEMONLY = False


def _dot(a, b, hi=False):
    prec = lax.Precision.HIGHEST if hi else lax.Precision.DEFAULT
    return jnp.dot(a, b, precision=prec, preferred_element_type=jnp.float32)


def _tc_body(xn_ref, idx_ref, meta_ref, emb_ref, wnb_ref, bn_ref, wmb_ref,
             bm_ref, fid_ref, g_ref, bta_ref, gs_ref, gb_ref, pxn_ref,
             pidx_ref, pmdn_ref, smdc_ref, esgn_ref,
             h0_ref, raw_ref, mask_ref, md_ref):
    if _MEMONLY:
        h0_ref[:, :WC] = emb_ref[...]
        raw_ref[...] = xn_ref[...] * 0.0
        mask_ref[...] = idx_ref[...].astype(jnp.float32)[:, 0:1] * jnp.zeros((TB, NF), jnp.float32)
        md_ref[...] = meta_ref[...][:, 0:1] * jnp.zeros((TB, 2 * NF), jnp.float32)
        return
    xn = xn_ref[...]
    meta = meta_ref[...]
    emb = emb_ref[...]
    idx = idx_ref[...]

    gelu = lambda t: 0.5 * t * (1.0 + lax.erf(t * 0.7071067811865476))
    h_num = gelu(_dot(xn, wnb_ref[...]) + bn_ref[...])
    h_cat = gelu(emb + _dot(meta, wmb_ref[...]) + bm_ref[...])
    h = jnp.concatenate([h_num, h_cat], axis=1) + fid_ref[...]

    Gs, Gb = gs_ref[...], gb_ref[...]
    mean_b = _dot(_dot(h, Gs), Gb)
    hc = h - mean_b
    rstd_g = lax.rsqrt(_dot(hc * hc, Gs) + 1e-5)
    h0_ref[...] = hc * _dot(rstd_g, Gb) * g_ref[...] + bta_ref[...]

    idxf = idx.astype(jnp.float32)
    raw_ref[...] = _dot(xn, pxn_ref[...], hi=True) + _dot(idxf, pidx_ref[...], hi=True)
    col = lax.broadcasted_iota(jnp.int32, (TB, NF), 1)
    mask_ref[...] = jnp.where(col < N_NUM, 1.0, 0.0)

    sgn = ((idx & 1) * 2 - 1).astype(jnp.float32)
    mA = _dot(xn, pmdn_ref[...], hi=True)
    mB = _dot(meta, smdc_ref[...], hi=True)
    sE = _dot(sgn, esgn_ref[...])
    lane = lax.broadcasted_iota(jnp.int32, (TB, 2 * NF), 1)
    is_ce = (lane >= 2 * N_NUM) & (lane % 2 == 0)
    md_ref[...] = jnp.where(is_ce, 0.5 + sE * 0.5 * (1.0 - mB), mA + mB)


def _tc_fused(xn2, idx, meta2, emb2, WnB, bnT, WmB, bmT, fidT, gamT, betT):
    grid = (B // TB,)
    row_spec = lambda w: pl.BlockSpec((TB, w), lambda i: (i, 0))
    full_spec = lambda r, c: pl.BlockSpec((r, c), lambda i: (0, 0))
    Gs, Gb, Pxn, Pidx, Pmdn, Smdc, Esgn = _MATS
    return pl.pallas_call(
        _tc_body,
        grid=grid,
        in_specs=[
            row_spec(NF), row_spec(N_CAT), row_spec(2 * N_CAT), row_spec(WC),
            full_spec(NF, WN), full_spec(1, WN),
            full_spec(2 * N_CAT, WC), full_spec(1, WC),
            full_spec(1, WT), full_spec(1, WT), full_spec(1, WT),
            full_spec(WT, NF), full_spec(NF, WT),
            full_spec(NF, NF), full_spec(N_CAT, NF),
            full_spec(NF, 2 * NF), full_spec(2 * N_CAT, 2 * NF),
            full_spec(N_CAT, 2 * NF),
        ],
        out_specs=[row_spec(WT), row_spec(NF), row_spec(NF), row_spec(2 * NF)],
        out_shape=[
            jax.ShapeDtypeStruct((B, WT), jnp.float32),
            jax.ShapeDtypeStruct((B, NF), jnp.float32),
            jax.ShapeDtypeStruct((B, NF), jnp.float32),
            jax.ShapeDtypeStruct((B, 2 * NF), jnp.float32),
        ],
        compiler_params=pltpu.CompilerParams(dimension_semantics=("parallel",)),
    )(xn2, idx, meta2, emb2, WnB, bnT, WmB, bmT, fidT, gamT, betT,
      Gs, Gb, Pxn, Pidx, Pmdn, Smdc, Esgn)


def kernel(x_numerical, x_categorical_idx, x_categorical_meta, W_num, b_num,
           table, W_meta, b_meta, feature_identity, gamma, beta):
    idx = x_categorical_idx.astype(jnp.int32)
    emb2 = _sc_gather(table, idx.reshape(-1)).reshape(B, WC)
    eye_n = jnp.eye(N_NUM, dtype=jnp.float32)
    eye_c = jnp.eye(N_CAT, dtype=jnp.float32)
    h0, raw, mask, md = _tc_fused(
        x_numerical.reshape(B, NF), idx,
        x_categorical_meta.reshape(B, 2 * N_CAT), emb2,
        jnp.kron(eye_n, W_num), jnp.tile(b_num, N_NUM).reshape(1, WN),
        jnp.kron(eye_c, W_meta), jnp.tile(b_meta, N_CAT).reshape(1, WC),
        feature_identity.reshape(1, WT),
        jnp.tile(gamma, NF).reshape(1, WT), jnp.tile(beta, NF).reshape(1, WT),
    )
    return (h0, raw, mask, md)


# speedup vs baseline: 1.1182x; 1.1182x over previous
import functools

import numpy as np
import jax
import jax.numpy as jnp
from jax import lax
from jax.experimental import pallas as pl
from jax.experimental.pallas import tpu as pltpu
from jax.experimental.pallas import tpu_sc as plsc

B, N_NUM, N_CAT, D = 16384, 13, 26, 32
NF = N_NUM + N_CAT
WN, WC, WT = N_NUM * D, N_CAT * D, NF * D

NC, NS = 2, 16
NW = NC * NS
ROWS = B * N_CAT
CHUNK = 128
CPW = ROWS // (NW * CHUNK)
ROWS_PW = ROWS // NW


def _sc_gather(table, idx_flat):
    idx2d = idx_flat.reshape(ROWS // CHUNK, CHUNK)
    mesh = plsc.VectorSubcoreMesh(
        core_axis_name="c", subcore_axis_name="s", num_cores=NC, num_subcores=NS
    )

    @functools.partial(
        pl.kernel,
        out_type=jax.ShapeDtypeStruct((ROWS, D), jnp.float32),
        mesh=mesh,
        scratch_types=[
            pltpu.VMEM((CPW, CHUNK), jnp.int32),
            pltpu.VMEM((CHUNK, D), jnp.float32),
            pltpu.SemaphoreType.DMA,
        ],
        compiler_params=pltpu.CompilerParams(use_tc_tiling_on_sc=False),
    )
    def gather_k(idx_hbm, table_hbm, out_hbm, idx_v, rows_v, sem):
        wid = lax.axis_index("s") * NC + lax.axis_index("c")
        pltpu.sync_copy(idx_hbm.at[pl.ds(wid * CPW, CPW)], idx_v)
        base = wid * ROWS_PW

        def step(j, carry):
            pltpu.async_copy(table_hbm.at[idx_v.at[j]], rows_v, sem).wait()
            pltpu.sync_copy(rows_v, out_hbm.at[pl.ds(base + j * CHUNK, CHUNK)])
            return carry

        lax.fori_loop(0, CPW, step, 0)

    return gather_k(idx2d, table)


def _build_mats():
    l_t = np.arange(WT)
    g_t = np.arange(NF)
    GsT = ((l_t[None, :] // D == g_t[:, None]).astype(np.float32) / D)
    GbT = (l_t[:, None] // D == g_t[None, :]).astype(np.float32)
    PrA = np.zeros((NF, NF), np.float32)
    for f in range(N_NUM):
        PrA[f, 3 * f] = 1.0
    PrB = np.zeros((NF, N_CAT), np.float32)
    for f in range(N_CAT):
        PrB[N_NUM + f, f] = 1.0
    M1a = np.zeros((2 * NF, NF), np.float32)
    for f in range(N_NUM):
        M1a[2 * f, 3 * f + 1] = 1.0
        M1a[2 * f + 1, 3 * f + 2] = 1.0
    M1b = np.zeros((2 * NF, 2 * N_CAT), np.float32)
    for j in range(2 * N_CAT):
        M1b[2 * N_NUM + j, j] = 1.0
    E1 = np.zeros((2 * NF, N_CAT), np.float32)
    for f in range(N_CAT):
        E1[2 * N_NUM + 2 * f, f] = 1.0
    return GsT, GbT, PrA, PrB, M1a, M1b, E1


_MATS = _build_mats()

TB = 512


def _dot(a, b, hi=False):
    prec = lax.Precision.HIGHEST if hi else lax.Precision.DEFAULT
    return jnp.dot(a, b, precision=prec, preferred_element_type=jnp.float32)


def _tc_body(xnT_ref, idxT_ref, metaT_ref, emb_ref, w1_ref, w2_ref, bias_ref,
             fid_ref, gam_ref, bet_ref, gs_ref, gb_ref, pra_ref, prb_ref,
             m1a_ref, m1b_ref, e1_ref,
             h0_ref, raw_ref, mask_ref, md_ref):
    xnT = xnT_ref[...]
    idxT = idxT_ref[...]
    metaT = metaT_ref[...]
    emb = emb_ref[...]

    gelu = lambda t: 0.5 * t * (1.0 + lax.erf(t * 0.7071067811865476))
    r_i = lax.broadcasted_iota(jnp.int32, (TB, TB), 0)
    c_i = lax.broadcasted_iota(jnp.int32, (TB, TB), 1)
    ident = jnp.where(r_i == c_i, 1.0, 0.0)
    embT = lax.dot_general(emb, ident, (((0,), (0,)), ((), ())),
                           precision=lax.Precision.DEFAULT,
                           preferred_element_type=jnp.float32)

    projN = _dot(w1_ref[...], xnT)
    projC = _dot(w2_ref[...], metaT)
    pre = jnp.concatenate([projN, projC + embT], axis=0) + bias_ref[...]
    h = gelu(pre) + fid_ref[...]

    GsT, GbT = gs_ref[...], gb_ref[...]
    mean_b = _dot(GbT, _dot(GsT, h))
    hc = h - mean_b
    rstd = lax.rsqrt(_dot(GsT, hc * hc) + 1e-5)
    h0_ref[...] = hc * _dot(GbT, rstd) * gam_ref[...] + bet_ref[...]

    idxfT = idxT.astype(jnp.float32)
    raw_ref[...] = _dot(pra_ref[...], xnT, hi=True) + _dot(prb_ref[...], idxfT, hi=True)
    row39 = lax.broadcasted_iota(jnp.int32, (NF, TB), 0)
    mask_ref[...] = jnp.where(row39 < N_NUM, 1.0, 0.0)

    sgnf = ((idxT & 1) * 2 - 1).astype(jnp.float32)
    r1 = _dot(m1a_ref[...], xnT, hi=True) + _dot(m1b_ref[...], metaT, hi=True)
    sE = _dot(e1_ref[...], sgnf)
    row78 = lax.broadcasted_iota(jnp.int32, (2 * NF, TB), 0)
    is_ce = (row78 >= 2 * N_NUM) & (row78 % 2 == 0)
    md_ref[...] = jnp.where(is_ce, 0.5 + sE * 0.5 * (1.0 - r1), r1)


def _tc_fused(xnT, idxT, metaT, emb2, W1, W2, biasT, fidT, gamT, betT):
    grid = (B // TB,)
    col_spec = lambda r: pl.BlockSpec((r, TB), lambda i: (0, i))
    full_spec = lambda r, c: pl.BlockSpec((r, c), lambda i: (0, 0))
    GsT, GbT, PrA, PrB, M1a, M1b, E1 = _MATS
    return pl.pallas_call(
        _tc_body,
        grid=grid,
        in_specs=[
            col_spec(NF), col_spec(N_CAT), col_spec(2 * N_CAT),
            pl.BlockSpec((TB, WC), lambda i: (i, 0)),
            full_spec(WN, NF), full_spec(WC, 2 * N_CAT), full_spec(WT, 1),
            full_spec(WT, 1), full_spec(WT, 1), full_spec(WT, 1),
            full_spec(NF, WT), full_spec(WT, NF),
            full_spec(NF, NF), full_spec(NF, N_CAT),
            full_spec(2 * NF, NF), full_spec(2 * NF, 2 * N_CAT),
            full_spec(2 * NF, N_CAT),
        ],
        out_specs=[col_spec(WT), col_spec(NF), col_spec(NF), col_spec(2 * NF)],
        out_shape=[
            jax.ShapeDtypeStruct((WT, B), jnp.float32),
            jax.ShapeDtypeStruct((NF, B), jnp.float32),
            jax.ShapeDtypeStruct((NF, B), jnp.float32),
            jax.ShapeDtypeStruct((2 * NF, B), jnp.float32),
        ],
        compiler_params=pltpu.CompilerParams(dimension_semantics=("parallel",)),
    )(xnT, idxT, metaT, emb2, W1, W2, biasT, fidT, gamT, betT,
      GsT, GbT, PrA, PrB, M1a, M1b, E1)


def kernel(x_numerical, x_categorical_idx, x_categorical_meta, W_num, b_num,
           table, W_meta, b_meta, feature_identity, gamma, beta):
    idx = x_categorical_idx.astype(jnp.int32)
    emb2 = _sc_gather(table, idx.reshape(-1)).reshape(B, WC)
    eye_n = jnp.eye(N_NUM, dtype=jnp.float32)
    eye_c = jnp.eye(N_CAT, dtype=jnp.float32)
    biasT = jnp.concatenate([jnp.tile(b_num, N_NUM), jnp.tile(b_meta, N_CAT)])
    h0T, rawT, maskT, mdT = _tc_fused(
        x_numerical.transpose(1, 2, 0).reshape(NF, B), idx.T,
        x_categorical_meta.transpose(1, 2, 0).reshape(2 * N_CAT, B), emb2,
        jnp.kron(eye_n, W_num).T, jnp.kron(eye_c, W_meta).T,
        biasT.reshape(WT, 1), feature_identity.reshape(WT, 1),
        jnp.tile(gamma, NF).reshape(WT, 1), jnp.tile(beta, NF).reshape(WT, 1),
    )
    h0 = h0T.reshape(NF, D, B).transpose(2, 0, 1)
    raw = rawT.T[:, :, None]
    md = mdT.reshape(NF, 2, B).transpose(2, 0, 1)
    return (h0, raw, maskT.T, md)


# speedup vs baseline: 1.2195x; 1.0905x over previous
import functools

import numpy as np
import jax
import jax.numpy as jnp
from jax import lax
from jax.experimental import pallas as pl
from jax.experimental.pallas import tpu as pltpu
from jax.experimental.pallas import tpu_sc as plsc

B, N_NUM, N_CAT, D = 16384, 13, 26, 32
NF = N_NUM + N_CAT
WN, WC, WT = N_NUM * D, N_CAT * D, NF * D

NC, NS = 2, 16
NW = NC * NS
ROWS = B * N_CAT
CHUNK = 128
CPW = ROWS // (NW * CHUNK)
ROWS_PW = ROWS // NW


def _sc_gather(table, idx_flat):
    idx2d = idx_flat.reshape(ROWS // CHUNK, CHUNK)
    mesh = plsc.VectorSubcoreMesh(
        core_axis_name="c", subcore_axis_name="s", num_cores=NC, num_subcores=NS
    )

    NBUF = 4

    @functools.partial(
        pl.kernel,
        out_type=jax.ShapeDtypeStruct((ROWS, D), jnp.float32),
        mesh=mesh,
        scratch_types=[
            pltpu.VMEM((CPW, CHUNK), jnp.int32),
            [pltpu.VMEM((CHUNK, D), jnp.float32) for _ in range(NBUF)],
            [pltpu.SemaphoreType.DMA for _ in range(NBUF)],
        ],
        compiler_params=pltpu.CompilerParams(use_tc_tiling_on_sc=False),
    )
    def gather_k(idx_hbm, table_hbm, out_hbm, idx_v, rows, sems):
        wid = lax.axis_index("s") * NC + lax.axis_index("c")
        pltpu.sync_copy(idx_hbm.at[pl.ds(wid * CPW, CPW)], idx_v)
        base = wid * ROWS_PW

        copies = [
            pltpu.async_copy(table_hbm.at[idx_v.at[j]], rows[j], sems[j])
            for j in range(NBUF)
        ]
        for j in range(CPW):
            b = j % NBUF
            copies[j].wait()
            pltpu.sync_copy(rows[b], out_hbm.at[pl.ds(base + j * CHUNK, CHUNK)])
            if j + NBUF < CPW:
                copies.append(
                    pltpu.async_copy(
                        table_hbm.at[idx_v.at[j + NBUF]], rows[b], sems[b]
                    )
                )

    return gather_k(idx2d, table)


def _build_mats():
    l_t = np.arange(WT)
    g_t = np.arange(NF)
    GsT = ((l_t[None, :] // D == g_t[:, None]).astype(np.float32) / D)
    GbT = (l_t[:, None] // D == g_t[None, :]).astype(np.float32)
    PrA = np.zeros((NF, NF), np.float32)
    for f in range(N_NUM):
        PrA[f, 3 * f] = 1.0
    PrB = np.zeros((NF, N_CAT), np.float32)
    for f in range(N_CAT):
        PrB[N_NUM + f, f] = 1.0
    M1a = np.zeros((2 * NF, NF), np.float32)
    for f in range(N_NUM):
        M1a[2 * f, 3 * f + 1] = 1.0
        M1a[2 * f + 1, 3 * f + 2] = 1.0
    M1b = np.zeros((2 * NF, 2 * N_CAT), np.float32)
    for j in range(2 * N_CAT):
        M1b[2 * N_NUM + j, j] = 1.0
    E1 = np.zeros((2 * NF, N_CAT), np.float32)
    for f in range(N_CAT):
        E1[2 * N_NUM + 2 * f, f] = 1.0
    return GsT, GbT, PrA, PrB, M1a, M1b, E1


_MATS = _build_mats()

TB = 512


def _dot(a, b, hi=False):
    prec = lax.Precision.HIGHEST if hi else lax.Precision.DEFAULT
    return jnp.dot(a, b, precision=prec, preferred_element_type=jnp.float32)


def _tc_body(xnT_ref, idxT_ref, metaT_ref, emb_ref, w1_ref, w2_ref, bias_ref,
             fid_ref, gam_ref, bet_ref, gs_ref, gb_ref, pra_ref, prb_ref,
             m1a_ref, m1b_ref, e1_ref,
             h0_ref, raw_ref, mask_ref, md_ref):
    xnT = xnT_ref[...]
    idxT = idxT_ref[...]
    metaT = metaT_ref[...]
    emb = emb_ref[...]

    gelu = lambda t: 0.5 * t * (1.0 + lax.erf(t * 0.7071067811865476))
    r_i = lax.broadcasted_iota(jnp.int32, (TB, TB), 0)
    c_i = lax.broadcasted_iota(jnp.int32, (TB, TB), 1)
    ident = jnp.where(r_i == c_i, 1.0, 0.0)
    embT = lax.dot_general(emb, ident, (((0,), (0,)), ((), ())),
                           precision=lax.Precision.DEFAULT,
                           preferred_element_type=jnp.float32)

    projN = _dot(w1_ref[...], xnT)
    projC = _dot(w2_ref[...], metaT)
    pre = jnp.concatenate([projN, projC + embT], axis=0) + bias_ref[...]
    h = gelu(pre) + fid_ref[...]

    GsT, GbT = gs_ref[...], gb_ref[...]
    mean_b = _dot(GbT, _dot(GsT, h))
    hc = h - mean_b
    rstd = lax.rsqrt(_dot(GsT, hc * hc) + 1e-5)
    h0_ref[...] = hc * _dot(GbT, rstd) * gam_ref[...] + bet_ref[...]

    idxfT = idxT.astype(jnp.float32)
    raw_ref[...] = _dot(pra_ref[...], xnT, hi=True) + _dot(prb_ref[...], idxfT, hi=True)
    row39 = lax.broadcasted_iota(jnp.int32, (NF, TB), 0)
    mask_ref[...] = jnp.where(row39 < N_NUM, 1.0, 0.0)

    sgnf = ((idxT & 1) * 2 - 1).astype(jnp.float32)
    r1 = _dot(m1a_ref[...], xnT, hi=True) + _dot(m1b_ref[...], metaT, hi=True)
    sE = _dot(e1_ref[...], sgnf)
    row78 = lax.broadcasted_iota(jnp.int32, (2 * NF, TB), 0)
    is_ce = (row78 >= 2 * N_NUM) & (row78 % 2 == 0)
    md_ref[...] = jnp.where(is_ce, 0.5 + sE * 0.5 * (1.0 - r1), r1)


def _tc_fused(xnT, idxT, metaT, emb2, W1, W2, biasT, fidT, gamT, betT):
    grid = (B // TB,)
    col_spec = lambda r: pl.BlockSpec((r, TB), lambda i: (0, i))
    full_spec = lambda r, c: pl.BlockSpec((r, c), lambda i: (0, 0))
    GsT, GbT, PrA, PrB, M1a, M1b, E1 = _MATS
    return pl.pallas_call(
        _tc_body,
        grid=grid,
        in_specs=[
            col_spec(NF), col_spec(N_CAT), col_spec(2 * N_CAT),
            pl.BlockSpec((TB, WC), lambda i: (i, 0)),
            full_spec(WN, NF), full_spec(WC, 2 * N_CAT), full_spec(WT, 1),
            full_spec(WT, 1), full_spec(WT, 1), full_spec(WT, 1),
            full_spec(NF, WT), full_spec(WT, NF),
            full_spec(NF, NF), full_spec(NF, N_CAT),
            full_spec(2 * NF, NF), full_spec(2 * NF, 2 * N_CAT),
            full_spec(2 * NF, N_CAT),
        ],
        out_specs=[col_spec(WT), col_spec(NF), col_spec(NF), col_spec(2 * NF)],
        out_shape=[
            jax.ShapeDtypeStruct((WT, B), jnp.float32),
            jax.ShapeDtypeStruct((NF, B), jnp.float32),
            jax.ShapeDtypeStruct((NF, B), jnp.float32),
            jax.ShapeDtypeStruct((2 * NF, B), jnp.float32),
        ],
        compiler_params=pltpu.CompilerParams(dimension_semantics=("parallel",)),
    )(xnT, idxT, metaT, emb2, W1, W2, biasT, fidT, gamT, betT,
      GsT, GbT, PrA, PrB, M1a, M1b, E1)


def kernel(x_numerical, x_categorical_idx, x_categorical_meta, W_num, b_num,
           table, W_meta, b_meta, feature_identity, gamma, beta):
    idx = x_categorical_idx.astype(jnp.int32)
    emb2 = _sc_gather(table, idx.reshape(-1)).reshape(B, WC)
    eye_n = jnp.eye(N_NUM, dtype=jnp.float32)
    eye_c = jnp.eye(N_CAT, dtype=jnp.float32)
    biasT = jnp.concatenate([jnp.tile(b_num, N_NUM), jnp.tile(b_meta, N_CAT)])
    h0T, rawT, maskT, mdT = _tc_fused(
        x_numerical.transpose(1, 2, 0).reshape(NF, B), idx.T,
        x_categorical_meta.transpose(1, 2, 0).reshape(2 * N_CAT, B), emb2,
        jnp.kron(eye_n, W_num).T, jnp.kron(eye_c, W_meta).T,
        biasT.reshape(WT, 1), feature_identity.reshape(WT, 1),
        jnp.tile(gamma, NF).reshape(WT, 1), jnp.tile(beta, NF).reshape(WT, 1),
    )
    h0 = h0T.reshape(NF, D, B).transpose(2, 0, 1)
    raw = rawT.T[:, :, None]
    md = mdT.reshape(NF, 2, B).transpose(2, 0, 1)
    return (h0, raw, maskT.T, md)
